# Initial kernel scaffold; baseline (speedup 1.0000x reference)
import jax
import jax.numpy as jnp
from jax.experimental import pallas as pl


def kernel(hidden_states, gate_w, Wg, Wu, Wd, Sg, Su, Sd):
    raise NotImplementedError("write your pallas kernel here")


# speedup vs baseline: 3.3725x; 3.3725x over previous
import functools

import jax
import jax.numpy as jnp
from jax import lax
from jax.experimental import pallas as pl
from jax.experimental.pallas import tpu as pltpu
from jax.experimental.pallas import tpu_sc as plsc

E = 64
K = 2
D = 768
DFF = 256
DSH = 512
T = 2048
R = 128
NT = T * K // R + E
NP = NT * R
NC, NS = 2, 16
NW = NC * NS

_HI = jax.lax.Precision.HIGHEST


def _gate_body(x_ref, gw_ref, i1_ref, i2_ref, w1_ref, w2_ref):
    x = x_ref[...]
    gw = gw_ref[...]
    logits = lax.dot_general(x, gw, (((1,), (1,)), ((), ())),
                             preferred_element_type=jnp.float32)
    m = jnp.max(logits, axis=-1, keepdims=True)
    p = jnp.exp(logits - m)
    probs = p / jnp.sum(p, axis=-1, keepdims=True)
    e_iota = lax.broadcasted_iota(jnp.int32, probs.shape, 1)
    v1 = jnp.max(probs, axis=-1, keepdims=True)
    i1 = jnp.min(jnp.where(probs == v1, e_iota, E), axis=-1, keepdims=True)
    masked = jnp.where(e_iota == i1, -jnp.inf, probs)
    v2 = jnp.max(masked, axis=-1, keepdims=True)
    i2 = jnp.min(jnp.where(masked == v2, e_iota, E), axis=-1, keepdims=True)
    denom = v1 + v2 + 1e-20
    i1_ref[...] = i1
    i2_ref[...] = i2
    w1_ref[...] = v1 / denom
    w2_ref[...] = v2 / denom


def _gate(x, gate_w):
    n = 8
    tb = T // n
    return pl.pallas_call(
        _gate_body,
        grid=(n,),
        in_specs=[
            pl.BlockSpec((tb, D), lambda i: (i, 0)),
            pl.BlockSpec((E, D), lambda i: (0, 0)),
        ],
        out_specs=[
            pl.BlockSpec((tb, 1), lambda i: (i, 0)),
            pl.BlockSpec((tb, 1), lambda i: (i, 0)),
            pl.BlockSpec((tb, 1), lambda i: (i, 0)),
            pl.BlockSpec((tb, 1), lambda i: (i, 0)),
        ],
        out_shape=[
            jax.ShapeDtypeStruct((T, 1), jnp.int32),
            jax.ShapeDtypeStruct((T, 1), jnp.int32),
            jax.ShapeDtypeStruct((T, 1), jnp.float32),
            jax.ShapeDtypeStruct((T, 1), jnp.float32),
        ],
    )(x, gate_w)


def _route_body(i0_ref, i1_ref, dest_ref, te_ref):
    ids = jnp.concatenate([i0_ref[...], i1_ref[...]], axis=0)
    rj, cj = ids.shape
    e3 = lax.broadcasted_iota(jnp.int32, (E, rj, cj), 0)
    oh = (ids[None, :, :] == e3).astype(jnp.float32)

    cu = lax.broadcasted_iota(jnp.int32, (cj, cj), 0)
    cl = lax.broadcasted_iota(jnp.int32, (cj, cj), 1)
    u_strict = (cu < cl).astype(jnp.float32)
    ones_c = jnp.ones((cj, cj), jnp.float32)
    r0_3 = lax.broadcasted_iota(jnp.int32, (E, rj, rj), 1)
    r1_3 = lax.broadcasted_iota(jnp.int32, (E, rj, rj), 2)
    l_strict3 = (r1_3 < r0_3).astype(jnp.float32)
    ones_r3 = jnp.ones((E, rj, rj), jnp.float32)

    term1 = lax.dot_general(oh, u_strict, (((2,), (0,)), ((), ())),
                            precision=_HI)
    rowsum3 = lax.dot_general(oh, ones_c, (((2,), (0,)), ((), ())),
                              precision=_HI)
    term2 = lax.dot_general(l_strict3, rowsum3, (((2,), (1,)), ((0,), (0,))),
                            precision=_HI)
    counts3 = lax.dot_general(ones_r3, rowsum3, (((2,), (1,)), ((0,), (0,))),
                              precision=_HI)

    n3 = jnp.floor((counts3 + (R - 1)) / R)
    el0 = lax.broadcasted_iota(jnp.int32, (E, E), 0)
    el1 = lax.broadcasted_iota(jnp.int32, (E, E), 1)
    e_strict = (el1 < el0).astype(jnp.float32)
    otiles3 = lax.dot_general(e_strict, n3, (((1,), (0,)), ((), ())),
                              precision=_HI)

    dest_f = jnp.sum(oh * (otiles3 * R + term1 + term2), axis=0)
    dest_ref[...] = jnp.round(dest_f).astype(jnp.int32)

    ends3 = otiles3 + n3
    i_iota = lax.broadcasted_iota(jnp.int32, (E, rj, cj), 2).astype(jnp.float32)
    te_f = jnp.sum((ends3 <= i_iota).astype(jnp.float32), axis=0)
    te = jnp.minimum(te_f, E - 1).astype(jnp.int32)
    te_ref[...] = te[:8, :]


def _route(ids0, ids1):
    return pl.pallas_call(
        _route_body,
        in_specs=[
            pl.BlockSpec((16, 128), lambda: (0, 0)),
            pl.BlockSpec((16, 128), lambda: (0, 0)),
        ],
        out_specs=[
            pl.BlockSpec((32, 128), lambda: (0, 0)),
            pl.BlockSpec((8, 128), lambda: (0, 0)),
        ],
        out_shape=[
            jax.ShapeDtypeStruct((32, 128), jnp.int32),
            jax.ShapeDtypeStruct((8, 128), jnp.int32),
        ],
    )(ids0, ids1)


def _dispatch_body(x_hbm, dest_hbm, xs_hbm, dest_v, rows_v, sem):
    wid = lax.axis_index("s") * NC + lax.axis_index("c")
    base = wid * (T * K // NW)
    pltpu.sync_copy(dest_hbm.at[pl.ds(base, 128)], dest_v)
    tok_base = lax.rem(base, T)
    pltpu.sync_copy(x_hbm.at[pl.ds(tok_base, 128)], rows_v)
    pltpu.async_copy(rows_v, xs_hbm.at[dest_v], sem).wait()


def _sc_dispatch(x, dest_flat):
    mesh = plsc.VectorSubcoreMesh(core_axis_name="c", subcore_axis_name="s",
                                  num_cores=NC, num_subcores=NS)
    f = pl.kernel(
        _dispatch_body,
        out_type=jax.ShapeDtypeStruct((NP, D), jnp.float32),
        mesh=mesh,
        scratch_types=[
            pltpu.VMEM((128,), jnp.int32),
            pltpu.VMEM((128, D), jnp.float32),
            pltpu.SemaphoreType.DMA,
        ],
    )
    return f(x, dest_flat)


def _expert_body(te_ref, xs_ref, wg_ref, wu_ref, wd_ref, ys_ref):
    xs = xs_ref[...]
    wg = wg_ref[0]
    wu = wu_ref[0]
    wd = wd_ref[0]
    g = lax.dot_general(xs, wg, (((1,), (1,)), ((), ())),
                        preferred_element_type=jnp.float32)
    u = lax.dot_general(xs, wu, (((1,), (1,)), ((), ())),
                        preferred_element_type=jnp.float32)
    h = (g * lax.logistic(g)) * u
    ys_ref[...] = lax.dot_general(h, wd, (((1,), (1,)), ((), ())),
                                  preferred_element_type=jnp.float32)


def _experts(te, xs, Wg, Wu, Wd):
    grid_spec = pltpu.PrefetchScalarGridSpec(
        num_scalar_prefetch=1,
        grid=(NT,),
        in_specs=[
            pl.BlockSpec((R, D), lambda i, te: (i, 0)),
            pl.BlockSpec((1, DFF, D), lambda i, te: (te[i], 0, 0)),
            pl.BlockSpec((1, DFF, D), lambda i, te: (te[i], 0, 0)),
            pl.BlockSpec((1, D, DFF), lambda i, te: (te[i], 0, 0)),
        ],
        out_specs=pl.BlockSpec((R, D), lambda i, te: (i, 0)),
    )
    return pl.pallas_call(
        _expert_body,
        grid_spec=grid_spec,
        out_shape=jax.ShapeDtypeStruct((NP, D), jnp.float32),
        compiler_params=pltpu.CompilerParams(
            dimension_semantics=("arbitrary",)),
    )(te, xs, Wg, Wu, Wd)


def _combine_body(ys_hbm, d0_hbm, d1_hbm, r0_hbm, r1_hbm, idx_v, rows_v, sem):
    wid = lax.axis_index("s") * NC + lax.axis_index("c")
    base = wid * (T // NW)
    pltpu.sync_copy(d0_hbm.at[pl.ds(base, T // NW)], idx_v)
    pltpu.async_copy(ys_hbm.at[idx_v], rows_v, sem).wait()
    pltpu.sync_copy(rows_v, r0_hbm.at[pl.ds(base, T // NW)])
    pltpu.sync_copy(d1_hbm.at[pl.ds(base, T // NW)], idx_v)
    pltpu.async_copy(ys_hbm.at[idx_v], rows_v, sem).wait()
    pltpu.sync_copy(rows_v, r1_hbm.at[pl.ds(base, T // NW)])


def _sc_combine(ys, d0, d1):
    mesh = plsc.VectorSubcoreMesh(core_axis_name="c", subcore_axis_name="s",
                                  num_cores=NC, num_subcores=NS)
    f = pl.kernel(
        _combine_body,
        out_type=(
            jax.ShapeDtypeStruct((T, D), jnp.float32),
            jax.ShapeDtypeStruct((T, D), jnp.float32),
        ),
        mesh=mesh,
        scratch_types=[
            pltpu.VMEM((T // NW,), jnp.int32),
            pltpu.VMEM((T // NW, D), jnp.float32),
            pltpu.SemaphoreType.DMA,
        ],
    )
    return f(ys, d0, d1)


def _final_body(x_ref, sg_ref, su_ref, sd_ref, r0_ref, r1_ref,
                w1_ref, w2_ref, y_ref):
    x = x_ref[...]
    g = lax.dot_general(x, sg_ref[...], (((1,), (1,)), ((), ())),
                        preferred_element_type=jnp.float32)
    u = lax.dot_general(x, su_ref[...], (((1,), (1,)), ((), ())),
                        preferred_element_type=jnp.float32)
    h = (g * lax.logistic(g)) * u
    sh = lax.dot_general(h, sd_ref[...], (((1,), (1,)), ((), ())),
                         preferred_element_type=jnp.float32)
    y_ref[...] = sh + w1_ref[...] * r0_ref[...] + w2_ref[...] * r1_ref[...]


def _final(x, Sg, Su, Sd, r0, r1, w1, w2):
    n = 8
    tb = T // n
    return pl.pallas_call(
        _final_body,
        grid=(n,),
        in_specs=[
            pl.BlockSpec((tb, D), lambda i: (i, 0)),
            pl.BlockSpec((DSH, D), lambda i: (0, 0)),
            pl.BlockSpec((DSH, D), lambda i: (0, 0)),
            pl.BlockSpec((D, DSH), lambda i: (0, 0)),
            pl.BlockSpec((tb, D), lambda i: (i, 0)),
            pl.BlockSpec((tb, D), lambda i: (i, 0)),
            pl.BlockSpec((tb, 1), lambda i: (i, 0)),
            pl.BlockSpec((tb, 1), lambda i: (i, 0)),
        ],
        out_specs=pl.BlockSpec((tb, D), lambda i: (i, 0)),
        out_shape=jax.ShapeDtypeStruct((T, D), jnp.float32),
    )(x, Sg, Su, Sd, r0, r1, w1, w2)


@jax.jit
def kernel(hidden_states, gate_w, Wg, Wu, Wd, Sg, Su, Sd):
    orig_shape = hidden_states.shape
    x = hidden_states.reshape(T, D)
    i1, i2, w1, w2 = _gate(x, gate_w)
    dest, te8 = _route(i1.reshape(16, 128), i2.reshape(16, 128))
    dest_flat = dest.reshape(T * K)
    te = te8[0]
    xs = _sc_dispatch(x, dest_flat)
    ys = _experts(te, xs, Wg, Wu, Wd)
    r0, r1 = _sc_combine(ys, dest_flat[:T], dest_flat[T:])
    y = _final(x, Sg, Su, Sd, r0, r1, w1, w2)
    return y.reshape(orig_shape)


# speedup vs baseline: 3.4151x; 1.0126x over previous
import functools

import jax
import jax.numpy as jnp
from jax import lax
from jax.experimental import pallas as pl
from jax.experimental.pallas import tpu as pltpu
from jax.experimental.pallas import tpu_sc as plsc

E = 64
K = 2
D = 768
DFF = 256
DSH = 512
T = 2048
R = 128
NT = T * K // R + E
NP = NT * R
NC, NS = 2, 16
NW = NC * NS

_HI = jax.lax.Precision.HIGHEST


def _gate_body(x_ref, gw_ref, i1_ref, i2_ref, w1_ref, w2_ref):
    x = x_ref[...]
    gw = gw_ref[...]
    logits = lax.dot_general(x, gw, (((1,), (1,)), ((), ())),
                             preferred_element_type=jnp.float32)
    m = jnp.max(logits, axis=-1, keepdims=True)
    p = jnp.exp(logits - m)
    probs = p / jnp.sum(p, axis=-1, keepdims=True)
    e_iota = lax.broadcasted_iota(jnp.int32, probs.shape, 1)
    v1 = jnp.max(probs, axis=-1, keepdims=True)
    i1 = jnp.min(jnp.where(probs == v1, e_iota, E), axis=-1, keepdims=True)
    masked = jnp.where(e_iota == i1, -jnp.inf, probs)
    v2 = jnp.max(masked, axis=-1, keepdims=True)
    i2 = jnp.min(jnp.where(masked == v2, e_iota, E), axis=-1, keepdims=True)
    denom = v1 + v2 + 1e-20
    i1_ref[...] = i1
    i2_ref[...] = i2
    w1_ref[...] = v1 / denom
    w2_ref[...] = v2 / denom


def _gate(x, gate_w):
    n = 8
    tb = T // n
    return pl.pallas_call(
        _gate_body,
        grid=(n,),
        in_specs=[
            pl.BlockSpec((tb, D), lambda i: (i, 0)),
            pl.BlockSpec((E, D), lambda i: (0, 0)),
        ],
        out_specs=[
            pl.BlockSpec((tb, 1), lambda i: (i, 0)),
            pl.BlockSpec((tb, 1), lambda i: (i, 0)),
            pl.BlockSpec((tb, 1), lambda i: (i, 0)),
            pl.BlockSpec((tb, 1), lambda i: (i, 0)),
        ],
        out_shape=[
            jax.ShapeDtypeStruct((T, 1), jnp.int32),
            jax.ShapeDtypeStruct((T, 1), jnp.int32),
            jax.ShapeDtypeStruct((T, 1), jnp.float32),
            jax.ShapeDtypeStruct((T, 1), jnp.float32),
        ],
    )(x, gate_w)


def _route_body(i0_ref, i1_ref, dest_ref, te_ref):
    ids = jnp.concatenate([i0_ref[...], i1_ref[...]], axis=0)
    rj, cj = ids.shape
    e3 = lax.broadcasted_iota(jnp.int32, (E, rj, cj), 0)
    oh = (ids[None, :, :] == e3).astype(jnp.float32)

    cu = lax.broadcasted_iota(jnp.int32, (cj, cj), 0)
    cl = lax.broadcasted_iota(jnp.int32, (cj, cj), 1)
    u_strict = (cu < cl).astype(jnp.float32)
    ones_c = jnp.ones((cj, cj), jnp.float32)
    r0_3 = lax.broadcasted_iota(jnp.int32, (E, rj, rj), 1)
    r1_3 = lax.broadcasted_iota(jnp.int32, (E, rj, rj), 2)
    l_strict3 = (r1_3 < r0_3).astype(jnp.float32)
    ones_r3 = jnp.ones((E, rj, rj), jnp.float32)

    term1 = lax.dot_general(oh, u_strict, (((2,), (0,)), ((), ())),
                            precision=_HI)
    rowsum3 = lax.dot_general(oh, ones_c, (((2,), (0,)), ((), ())),
                              precision=_HI)
    term2 = lax.dot_general(l_strict3, rowsum3, (((2,), (1,)), ((0,), (0,))),
                            precision=_HI)
    counts3 = lax.dot_general(ones_r3, rowsum3, (((2,), (1,)), ((0,), (0,))),
                              precision=_HI)

    n3 = jnp.floor((counts3 + (R - 1)) / R)
    el0 = lax.broadcasted_iota(jnp.int32, (E, E), 0)
    el1 = lax.broadcasted_iota(jnp.int32, (E, E), 1)
    e_strict = (el1 < el0).astype(jnp.float32)
    otiles3 = lax.dot_general(e_strict, n3, (((1,), (0,)), ((), ())),
                              precision=_HI)

    dest_f = jnp.sum(oh * (otiles3 * R + term1 + term2), axis=0)
    dest_ref[...] = jnp.round(dest_f).astype(jnp.int32)

    ends3 = otiles3 + n3
    i_iota = lax.broadcasted_iota(jnp.int32, (E, rj, cj), 2).astype(jnp.float32)
    te_f = jnp.sum((ends3 <= i_iota).astype(jnp.float32), axis=0)
    te = jnp.minimum(te_f, E - 1).astype(jnp.int32)
    te_ref[...] = te[:8, :]


def _route(ids0, ids1):
    return pl.pallas_call(
        _route_body,
        in_specs=[
            pl.BlockSpec((16, 128), lambda: (0, 0)),
            pl.BlockSpec((16, 128), lambda: (0, 0)),
        ],
        out_specs=[
            pl.BlockSpec((32, 128), lambda: (0, 0)),
            pl.BlockSpec((8, 128), lambda: (0, 0)),
        ],
        out_shape=[
            jax.ShapeDtypeStruct((32, 128), jnp.int32),
            jax.ShapeDtypeStruct((8, 128), jnp.int32),
        ],
    )(ids0, ids1)


def _dispatch_body(x_hbm, dest_hbm, xs_hbm, dest_v, rows_v, sem):
    wid = lax.axis_index("s") * NC + lax.axis_index("c")
    base = wid * (T * K // NW)
    pltpu.sync_copy(dest_hbm.at[pl.ds(base, 128)], dest_v)
    tok_base = lax.rem(base, T)
    pltpu.sync_copy(x_hbm.at[pl.ds(tok_base, 128)], rows_v)
    pltpu.async_copy(rows_v, xs_hbm.at[dest_v], sem).wait()


def _sc_dispatch(x, dest_flat):
    mesh = plsc.VectorSubcoreMesh(core_axis_name="c", subcore_axis_name="s",
                                  num_cores=NC, num_subcores=NS)
    f = pl.kernel(
        _dispatch_body,
        out_type=jax.ShapeDtypeStruct((NP, D), jnp.float32),
        mesh=mesh,
        scratch_types=[
            pltpu.VMEM((128,), jnp.int32),
            pltpu.VMEM((128, D), jnp.float32),
            pltpu.SemaphoreType.DMA,
        ],
    )
    return f(x, dest_flat)


def _expert_body(te_ref, xs_ref, wg_ref, wu_ref, wd_ref, ys_ref):
    xs = xs_ref[...]
    wg = wg_ref[0]
    wu = wu_ref[0]
    wd = wd_ref[0]
    g = lax.dot_general(xs, wg, (((1,), (1,)), ((), ())),
                        preferred_element_type=jnp.float32)
    u = lax.dot_general(xs, wu, (((1,), (1,)), ((), ())),
                        preferred_element_type=jnp.float32)
    h = (g * lax.logistic(g)) * u
    ys_ref[...] = lax.dot_general(h, wd, (((1,), (1,)), ((), ())),
                                  preferred_element_type=jnp.float32)


def _experts(te, xs, Wg, Wu, Wd):
    grid_spec = pltpu.PrefetchScalarGridSpec(
        num_scalar_prefetch=1,
        grid=(NT,),
        in_specs=[
            pl.BlockSpec((R, D), lambda i, te: (i, 0)),
            pl.BlockSpec((1, DFF, D), lambda i, te: (te[i], 0, 0)),
            pl.BlockSpec((1, DFF, D), lambda i, te: (te[i], 0, 0)),
            pl.BlockSpec((1, D, DFF), lambda i, te: (te[i], 0, 0)),
        ],
        out_specs=pl.BlockSpec((R, D), lambda i, te: (i, 0)),
    )
    return pl.pallas_call(
        _expert_body,
        grid_spec=grid_spec,
        out_shape=jax.ShapeDtypeStruct((NP, D), jnp.float32),
        compiler_params=pltpu.CompilerParams(
            dimension_semantics=("arbitrary",)),
    )(te, xs, Wg, Wu, Wd)


def _combine_body(ys_hbm, d0_hbm, d1_hbm, r0_hbm, r1_hbm, idx_v, rows_v, sem):
    wid = lax.axis_index("s") * NC + lax.axis_index("c")
    base = wid * (T // NW)
    pltpu.sync_copy(d0_hbm.at[pl.ds(base, T // NW)], idx_v)
    pltpu.async_copy(ys_hbm.at[idx_v], rows_v, sem).wait()
    pltpu.sync_copy(rows_v, r0_hbm.at[pl.ds(base, T // NW)])
    pltpu.sync_copy(d1_hbm.at[pl.ds(base, T // NW)], idx_v)
    pltpu.async_copy(ys_hbm.at[idx_v], rows_v, sem).wait()
    pltpu.sync_copy(rows_v, r1_hbm.at[pl.ds(base, T // NW)])


def _sc_combine(ys, d0, d1):
    mesh = plsc.VectorSubcoreMesh(core_axis_name="c", subcore_axis_name="s",
                                  num_cores=NC, num_subcores=NS)
    f = pl.kernel(
        _combine_body,
        out_type=(
            jax.ShapeDtypeStruct((T, D), jnp.float32),
            jax.ShapeDtypeStruct((T, D), jnp.float32),
        ),
        mesh=mesh,
        scratch_types=[
            pltpu.VMEM((T // NW,), jnp.int32),
            pltpu.VMEM((T // NW, D), jnp.float32),
            pltpu.SemaphoreType.DMA,
        ],
    )
    return f(ys, d0, d1)


def _shared_body(x_ref, sg_ref, su_ref, sd_ref, sh_ref):
    x = x_ref[...]
    g = lax.dot_general(x, sg_ref[...], (((1,), (1,)), ((), ())),
                        preferred_element_type=jnp.float32)
    u = lax.dot_general(x, su_ref[...], (((1,), (1,)), ((), ())),
                        preferred_element_type=jnp.float32)
    h = (g * lax.logistic(g)) * u
    sh_ref[...] = lax.dot_general(h, sd_ref[...], (((1,), (1,)), ((), ())),
                                  preferred_element_type=jnp.float32)


def _shared(x, Sg, Su, Sd):
    n = 8
    tb = T // n
    return pl.pallas_call(
        _shared_body,
        grid=(n,),
        in_specs=[
            pl.BlockSpec((tb, D), lambda i: (i, 0)),
            pl.BlockSpec((DSH, D), lambda i: (0, 0)),
            pl.BlockSpec((DSH, D), lambda i: (0, 0)),
            pl.BlockSpec((D, DSH), lambda i: (0, 0)),
        ],
        out_specs=pl.BlockSpec((tb, D), lambda i: (i, 0)),
        out_shape=jax.ShapeDtypeStruct((T, D), jnp.float32),
    )(x, Sg, Su, Sd)


def _mix_body(sh_ref, r0_ref, r1_ref, w1_ref, w2_ref, y_ref):
    y_ref[...] = (sh_ref[...] + w1_ref[...] * r0_ref[...]
                  + w2_ref[...] * r1_ref[...])


def _mix(sh, r0, r1, w1, w2):
    n = 8
    tb = T // n
    return pl.pallas_call(
        _mix_body,
        grid=(n,),
        in_specs=[
            pl.BlockSpec((tb, D), lambda i: (i, 0)),
            pl.BlockSpec((tb, D), lambda i: (i, 0)),
            pl.BlockSpec((tb, D), lambda i: (i, 0)),
            pl.BlockSpec((tb, 1), lambda i: (i, 0)),
            pl.BlockSpec((tb, 1), lambda i: (i, 0)),
        ],
        out_specs=pl.BlockSpec((tb, D), lambda i: (i, 0)),
        out_shape=jax.ShapeDtypeStruct((T, D), jnp.float32),
    )(sh, r0, r1, w1, w2)


@jax.jit
def kernel(hidden_states, gate_w, Wg, Wu, Wd, Sg, Su, Sd):
    orig_shape = hidden_states.shape
    x = hidden_states.reshape(T, D)
    i1, i2, w1, w2 = _gate(x, gate_w)
    dest, te8 = _route(i1.reshape(16, 128), i2.reshape(16, 128))
    dest_flat = dest.reshape(T * K)
    te = te8[0]
    xs = _sc_dispatch(x, dest_flat)
    sh = _shared(x, Sg, Su, Sd)
    ys = _experts(te, xs, Wg, Wu, Wd)
    r0, r1 = _sc_combine(ys, dest_flat[:T], dest_flat[T:])
    y = _mix(sh, r0, r1, w1, w2)
    return y.reshape(orig_shape)


# speedup vs baseline: 3.4713x; 1.0164x over previous
import functools

import jax
import jax.numpy as jnp
from jax import lax
from jax.experimental import pallas as pl
from jax.experimental.pallas import tpu as pltpu
from jax.experimental.pallas import tpu_sc as plsc

E = 64
K = 2
D = 768
DFF = 256
DSH = 512
T = 2048
R = 128
NT = T * K // R + E
NP = NT * R
NC, NS = 2, 16
NW = NC * NS

_HI = jax.lax.Precision.HIGHEST


def _gate_body(x_ref, gw_ref, i1_ref, i2_ref, w1_ref, w2_ref):
    x = x_ref[...]
    gw = gw_ref[...]
    logits = lax.dot_general(x, gw, (((1,), (1,)), ((), ())),
                             preferred_element_type=jnp.float32)
    m = jnp.max(logits, axis=-1, keepdims=True)
    p = jnp.exp(logits - m)
    probs = p / jnp.sum(p, axis=-1, keepdims=True)
    e_iota = lax.broadcasted_iota(jnp.int32, probs.shape, 1)
    v1 = jnp.max(probs, axis=-1, keepdims=True)
    i1 = jnp.min(jnp.where(probs == v1, e_iota, E), axis=-1, keepdims=True)
    masked = jnp.where(e_iota == i1, -jnp.inf, probs)
    v2 = jnp.max(masked, axis=-1, keepdims=True)
    i2 = jnp.min(jnp.where(masked == v2, e_iota, E), axis=-1, keepdims=True)
    denom = v1 + v2 + 1e-20
    i1_ref[...] = i1
    i2_ref[...] = i2
    w1_ref[...] = v1 / denom
    w2_ref[...] = v2 / denom


def _gate(x, gate_w):
    n = 8
    tb = T // n
    return pl.pallas_call(
        _gate_body,
        grid=(n,),
        in_specs=[
            pl.BlockSpec((tb, D), lambda i: (i, 0)),
            pl.BlockSpec((E, D), lambda i: (0, 0)),
        ],
        out_specs=[
            pl.BlockSpec((tb, 1), lambda i: (i, 0)),
            pl.BlockSpec((tb, 1), lambda i: (i, 0)),
            pl.BlockSpec((tb, 1), lambda i: (i, 0)),
            pl.BlockSpec((tb, 1), lambda i: (i, 0)),
        ],
        out_shape=[
            jax.ShapeDtypeStruct((T, 1), jnp.int32),
            jax.ShapeDtypeStruct((T, 1), jnp.int32),
            jax.ShapeDtypeStruct((T, 1), jnp.float32),
            jax.ShapeDtypeStruct((T, 1), jnp.float32),
        ],
    )(x, gate_w)


def _route_body(i0_ref, i1_ref, dest_ref, te_ref):
    ids = jnp.concatenate([i0_ref[...], i1_ref[...]], axis=0)
    rj, cj = ids.shape
    e3 = lax.broadcasted_iota(jnp.int32, (E, rj, cj), 0)
    bf = jnp.bfloat16
    f32 = jnp.float32
    oh = (ids[None, :, :] == e3).astype(f32)
    ohb = oh.astype(bf)

    cu = lax.broadcasted_iota(jnp.int32, (cj, cj), 0)
    cl = lax.broadcasted_iota(jnp.int32, (cj, cj), 1)
    u_strict = (cu < cl).astype(bf)
    ones_c = jnp.ones((cj, cj), bf)
    r0_3 = lax.broadcasted_iota(jnp.int32, (E, rj, rj), 1)
    r1_3 = lax.broadcasted_iota(jnp.int32, (E, rj, rj), 2)
    l_strict3 = (r1_3 < r0_3).astype(bf)
    ones_r3 = jnp.ones((E, rj, rj), bf)

    term1 = lax.dot_general(ohb, u_strict, (((2,), (0,)), ((), ())),
                            preferred_element_type=f32)
    rowsum3 = lax.dot_general(ohb, ones_c, (((2,), (0,)), ((), ())),
                              preferred_element_type=f32).astype(bf)
    term2 = lax.dot_general(l_strict3, rowsum3, (((2,), (1,)), ((0,), (0,))),
                            preferred_element_type=f32)
    counts3 = lax.dot_general(ones_r3, rowsum3, (((2,), (1,)), ((0,), (0,))),
                              preferred_element_type=f32)

    n3 = jnp.floor((counts3 + (R - 1)) / R)
    el0 = lax.broadcasted_iota(jnp.int32, (E, E), 0)
    el1 = lax.broadcasted_iota(jnp.int32, (E, E), 1)
    e_strict = (el1 < el0).astype(bf)
    otiles3 = lax.dot_general(e_strict, n3.astype(bf), (((1,), (0,)), ((), ())),
                              preferred_element_type=f32)

    dest_f = jnp.sum(oh * (otiles3 * R + term1 + term2), axis=0)
    dest_ref[...] = jnp.round(dest_f).astype(jnp.int32)

    ends3 = otiles3 + n3
    i_iota = lax.broadcasted_iota(jnp.int32, (E, rj, cj), 2).astype(jnp.float32)
    te_f = jnp.sum((ends3 <= i_iota).astype(jnp.float32), axis=0)
    te = jnp.minimum(te_f, E - 1).astype(jnp.int32)
    te_ref[...] = te[:8, :]


def _route(ids0, ids1):
    return pl.pallas_call(
        _route_body,
        in_specs=[
            pl.BlockSpec((16, 128), lambda: (0, 0)),
            pl.BlockSpec((16, 128), lambda: (0, 0)),
        ],
        out_specs=[
            pl.BlockSpec((32, 128), lambda: (0, 0)),
            pl.BlockSpec((8, 128), lambda: (0, 0)),
        ],
        out_shape=[
            jax.ShapeDtypeStruct((32, 128), jnp.int32),
            jax.ShapeDtypeStruct((8, 128), jnp.int32),
        ],
    )(ids0, ids1)


def _dispatch_body(x_hbm, dest_hbm, xs_hbm, dest_v, rows_v, sem):
    wid = lax.axis_index("s") * NC + lax.axis_index("c")
    base = wid * (T * K // NW)
    pltpu.sync_copy(dest_hbm.at[pl.ds(base, 128)], dest_v)
    tok_base = lax.rem(base, T)
    pltpu.sync_copy(x_hbm.at[pl.ds(tok_base, 128)], rows_v)
    pltpu.async_copy(rows_v, xs_hbm.at[dest_v], sem).wait()


def _sc_dispatch(x, dest_flat):
    mesh = plsc.VectorSubcoreMesh(core_axis_name="c", subcore_axis_name="s",
                                  num_cores=NC, num_subcores=NS)
    f = pl.kernel(
        _dispatch_body,
        out_type=jax.ShapeDtypeStruct((NP, D), jnp.float32),
        mesh=mesh,
        scratch_types=[
            pltpu.VMEM((128,), jnp.int32),
            pltpu.VMEM((128, D), jnp.float32),
            pltpu.SemaphoreType.DMA,
        ],
    )
    return f(x, dest_flat)


def _expert_body(te_ref, xs_ref, wg_ref, wu_ref, wd_ref, ys_ref):
    xs = xs_ref[...].astype(jnp.bfloat16)
    wg = wg_ref[0].astype(jnp.bfloat16)
    wu = wu_ref[0].astype(jnp.bfloat16)
    wd = wd_ref[0].astype(jnp.bfloat16)
    g = lax.dot_general(xs, wg, (((1,), (1,)), ((), ())),
                        preferred_element_type=jnp.float32)
    u = lax.dot_general(xs, wu, (((1,), (1,)), ((), ())),
                        preferred_element_type=jnp.float32)
    h = ((g * lax.logistic(g)) * u).astype(jnp.bfloat16)
    ys_ref[...] = lax.dot_general(h, wd, (((1,), (1,)), ((), ())),
                                  preferred_element_type=jnp.float32)


def _experts(te, xs, Wg, Wu, Wd):
    grid_spec = pltpu.PrefetchScalarGridSpec(
        num_scalar_prefetch=1,
        grid=(NT,),
        in_specs=[
            pl.BlockSpec((R, D), lambda i, te: (i, 0)),
            pl.BlockSpec((1, DFF, D), lambda i, te: (te[i], 0, 0)),
            pl.BlockSpec((1, DFF, D), lambda i, te: (te[i], 0, 0)),
            pl.BlockSpec((1, D, DFF), lambda i, te: (te[i], 0, 0)),
        ],
        out_specs=pl.BlockSpec((R, D), lambda i, te: (i, 0)),
    )
    return pl.pallas_call(
        _expert_body,
        grid_spec=grid_spec,
        out_shape=jax.ShapeDtypeStruct((NP, D), jnp.float32),
        compiler_params=pltpu.CompilerParams(
            dimension_semantics=("arbitrary",)),
    )(te, xs, Wg, Wu, Wd)


def _combine_body(ys_hbm, d0_hbm, d1_hbm, r0_hbm, r1_hbm, idx_v, rows_v, sem):
    wid = lax.axis_index("s") * NC + lax.axis_index("c")
    base = wid * (T // NW)
    pltpu.sync_copy(d0_hbm.at[pl.ds(base, T // NW)], idx_v)
    pltpu.async_copy(ys_hbm.at[idx_v], rows_v, sem).wait()
    pltpu.sync_copy(rows_v, r0_hbm.at[pl.ds(base, T // NW)])
    pltpu.sync_copy(d1_hbm.at[pl.ds(base, T // NW)], idx_v)
    pltpu.async_copy(ys_hbm.at[idx_v], rows_v, sem).wait()
    pltpu.sync_copy(rows_v, r1_hbm.at[pl.ds(base, T // NW)])


def _sc_combine(ys, d0, d1):
    mesh = plsc.VectorSubcoreMesh(core_axis_name="c", subcore_axis_name="s",
                                  num_cores=NC, num_subcores=NS)
    f = pl.kernel(
        _combine_body,
        out_type=(
            jax.ShapeDtypeStruct((T, D), jnp.float32),
            jax.ShapeDtypeStruct((T, D), jnp.float32),
        ),
        mesh=mesh,
        scratch_types=[
            pltpu.VMEM((T // NW,), jnp.int32),
            pltpu.VMEM((T // NW, D), jnp.float32),
            pltpu.SemaphoreType.DMA,
        ],
    )
    return f(ys, d0, d1)


def _shared_body(x_ref, sg_ref, su_ref, sd_ref, sh_ref):
    x = x_ref[...].astype(jnp.bfloat16)
    sg = sg_ref[...].astype(jnp.bfloat16)
    su = su_ref[...].astype(jnp.bfloat16)
    sd = sd_ref[...].astype(jnp.bfloat16)
    g = lax.dot_general(x, sg, (((1,), (1,)), ((), ())),
                        preferred_element_type=jnp.float32)
    u = lax.dot_general(x, su, (((1,), (1,)), ((), ())),
                        preferred_element_type=jnp.float32)
    h = ((g * lax.logistic(g)) * u).astype(jnp.bfloat16)
    sh_ref[...] = lax.dot_general(h, sd, (((1,), (1,)), ((), ())),
                                  preferred_element_type=jnp.float32)


def _shared(x, Sg, Su, Sd):
    n = 8
    tb = T // n
    return pl.pallas_call(
        _shared_body,
        grid=(n,),
        in_specs=[
            pl.BlockSpec((tb, D), lambda i: (i, 0)),
            pl.BlockSpec((DSH, D), lambda i: (0, 0)),
            pl.BlockSpec((DSH, D), lambda i: (0, 0)),
            pl.BlockSpec((D, DSH), lambda i: (0, 0)),
        ],
        out_specs=pl.BlockSpec((tb, D), lambda i: (i, 0)),
        out_shape=jax.ShapeDtypeStruct((T, D), jnp.float32),
    )(x, Sg, Su, Sd)


def _mix_body(sh_ref, r0_ref, r1_ref, w1_ref, w2_ref, y_ref):
    y_ref[...] = (sh_ref[...] + w1_ref[...] * r0_ref[...]
                  + w2_ref[...] * r1_ref[...])


def _mix(sh, r0, r1, w1, w2):
    n = 8
    tb = T // n
    return pl.pallas_call(
        _mix_body,
        grid=(n,),
        in_specs=[
            pl.BlockSpec((tb, D), lambda i: (i, 0)),
            pl.BlockSpec((tb, D), lambda i: (i, 0)),
            pl.BlockSpec((tb, D), lambda i: (i, 0)),
            pl.BlockSpec((tb, 1), lambda i: (i, 0)),
            pl.BlockSpec((tb, 1), lambda i: (i, 0)),
        ],
        out_specs=pl.BlockSpec((tb, D), lambda i: (i, 0)),
        out_shape=jax.ShapeDtypeStruct((T, D), jnp.float32),
    )(sh, r0, r1, w1, w2)


@jax.jit
def kernel(hidden_states, gate_w, Wg, Wu, Wd, Sg, Su, Sd):
    orig_shape = hidden_states.shape
    x = hidden_states.reshape(T, D)
    i1, i2, w1, w2 = _gate(x, gate_w)
    dest, te8 = _route(i1.reshape(16, 128), i2.reshape(16, 128))
    dest_flat = dest.reshape(T * K)
    te = te8[0]
    xs = _sc_dispatch(x, dest_flat)
    sh = _shared(x, Sg, Su, Sd)
    ys = _experts(te, xs, Wg, Wu, Wd)
    r0, r1 = _sc_combine(ys, dest_flat[:T], dest_flat[T:])
    y = _mix(sh, r0, r1, w1, w2)
    return y.reshape(orig_shape)
